# baseline (device time: 52786 ns/iter reference)
import jax
import jax.numpy as jnp
from jax import lax
from jax.experimental import pallas as pl
from jax.experimental.pallas import tpu as pltpu

_N_CHUNKS = 16


def kernel(x, pi):
    rows = x.shape[1] // _N_CHUNKS
    x = pltpu.with_memory_space_constraint(x, pltpu.MemorySpace.HBM)

    def body(pi_ref, x_hbm, out_ref, x_vmem, send_buf, local_sems,
             send_sems, recv_sems):
        my_x = lax.axis_index("x")
        my_y = lax.axis_index("y")
        my_z = lax.axis_index("z")
        dst_y = pi_ref[my_y]
        other_y = 1 - my_y

        local_copies = []
        for c in range(_N_CHUNKS):
            sl = pl.ds(c * rows, rows)
            cp = pltpu.make_async_copy(
                x_hbm.at[:, sl, :], x_vmem.at[:, sl, :], local_sems.at[c]
            )
            cp.start()
            local_copies.append(cp)

        barrier = pltpu.get_barrier_semaphore()
        pl.semaphore_signal(
            barrier,
            inc=1,
            device_id=(my_x, other_y, my_z),
            device_id_type=pl.DeviceIdType.MESH,
        )
        pl.semaphore_wait(barrier, 1)

        @pl.when(dst_y == my_y)
        def _():
            for c in range(_N_CHUNKS):
                local_copies[c].wait()
            send_buf[...] = x_vmem[...].astype(jnp.bfloat16)
            copy = pltpu.make_async_copy(send_buf, out_ref, send_sems.at[0])
            copy.start()
            copy.wait()

        @pl.when(dst_y != my_y)
        def _():
            rdmas = []
            for c in range(_N_CHUNKS):
                sl = pl.ds(c * rows, rows)
                local_copies[c].wait()
                send_buf[0, sl, :] = x_vmem[0, sl, :].astype(jnp.bfloat16)
                rdma = pltpu.make_async_remote_copy(
                    src_ref=send_buf.at[:, sl, :],
                    dst_ref=out_ref.at[:, sl, :],
                    send_sem=send_sems.at[c],
                    recv_sem=recv_sems.at[c],
                    device_id=(my_x, dst_y, my_z),
                    device_id_type=pl.DeviceIdType.MESH,
                )
                rdma.start()
                rdmas.append(rdma)
            for rdma in rdmas:
                rdma.wait()

    return pl.pallas_call(
        body,
        out_shape=jax.ShapeDtypeStruct(x.shape, jnp.bfloat16),
        in_specs=[
            pl.BlockSpec(memory_space=pltpu.SMEM),
            pl.BlockSpec(memory_space=pl.ANY),
        ],
        out_specs=pl.BlockSpec(memory_space=pl.ANY),
        scratch_shapes=[
            pltpu.VMEM(x.shape, x.dtype),
            pltpu.VMEM(x.shape, jnp.bfloat16),
            pltpu.SemaphoreType.DMA((_N_CHUNKS,)),
            pltpu.SemaphoreType.DMA((_N_CHUNKS,)),
            pltpu.SemaphoreType.DMA((_N_CHUNKS,)),
        ],
        compiler_params=pltpu.CompilerParams(collective_id=0),
    )(pi, x)


# device time: 51855 ns/iter; 1.0180x vs baseline; 1.0180x over previous
import jax
import jax.numpy as jnp
from jax import lax
from jax.experimental import pallas as pl
from jax.experimental.pallas import tpu as pltpu

_N_CHUNKS = 8


def kernel(x, pi):
    rows = x.shape[1] // _N_CHUNKS
    x = pltpu.with_memory_space_constraint(x, pltpu.MemorySpace.HBM)
    pi = pltpu.with_memory_space_constraint(pi, pltpu.MemorySpace.HBM)

    def body(pi_hbm, x_hbm, out_ref, pi_smem, x_vmem, send_buf, pi_sem,
             local_sems, send_sems, recv_sems):
        my_x = lax.axis_index("x")
        my_y = lax.axis_index("y")
        my_z = lax.axis_index("z")
        other_y = 1 - my_y

        pi_copy = pltpu.make_async_copy(pi_hbm, pi_smem, pi_sem)
        pi_copy.start()

        local_copies = []
        for c in range(_N_CHUNKS):
            sl = pl.ds(c * rows, rows)
            cp = pltpu.make_async_copy(
                x_hbm.at[:, sl, :], x_vmem.at[:, sl, :], local_sems.at[c]
            )
            cp.start()
            local_copies.append(cp)

        barrier = pltpu.get_barrier_semaphore()
        pl.semaphore_signal(
            barrier,
            inc=1,
            device_id=(my_x, other_y, my_z),
            device_id_type=pl.DeviceIdType.MESH,
        )
        pl.semaphore_wait(barrier, 1)

        pi_copy.wait()
        dst_y = pi_smem[my_y]

        @pl.when(dst_y == my_y)
        def _():
            for c in range(_N_CHUNKS):
                local_copies[c].wait()
            send_buf[...] = x_vmem[...].astype(jnp.bfloat16)
            copy = pltpu.make_async_copy(send_buf, out_ref, send_sems.at[0])
            copy.start()
            copy.wait()

        @pl.when(dst_y != my_y)
        def _():
            rdmas = []
            for c in range(_N_CHUNKS):
                sl = pl.ds(c * rows, rows)
                local_copies[c].wait()
                send_buf[0, sl, :] = x_vmem[0, sl, :].astype(jnp.bfloat16)
                rdma = pltpu.make_async_remote_copy(
                    src_ref=send_buf.at[:, sl, :],
                    dst_ref=out_ref.at[:, sl, :],
                    send_sem=send_sems.at[c],
                    recv_sem=recv_sems.at[c],
                    device_id=(my_x, dst_y, my_z),
                    device_id_type=pl.DeviceIdType.MESH,
                )
                rdma.start()
                rdmas.append(rdma)
            for rdma in rdmas:
                rdma.wait()

    return pl.pallas_call(
        body,
        out_shape=jax.ShapeDtypeStruct(x.shape, jnp.bfloat16),
        in_specs=[
            pl.BlockSpec(memory_space=pl.ANY),
            pl.BlockSpec(memory_space=pl.ANY),
        ],
        out_specs=pl.BlockSpec(memory_space=pl.ANY),
        scratch_shapes=[
            pltpu.SMEM(pi.shape, pi.dtype),
            pltpu.VMEM(x.shape, x.dtype),
            pltpu.VMEM(x.shape, jnp.bfloat16),
            pltpu.SemaphoreType.DMA,
            pltpu.SemaphoreType.DMA((_N_CHUNKS,)),
            pltpu.SemaphoreType.DMA((_N_CHUNKS,)),
            pltpu.SemaphoreType.DMA((_N_CHUNKS,)),
        ],
        compiler_params=pltpu.CompilerParams(collective_id=0),
    )(pi, x)


# device time: 51685 ns/iter; 1.0213x vs baseline; 1.0033x over previous
import jax
import jax.numpy as jnp
from jax import lax
from jax.experimental import pallas as pl
from jax.experimental.pallas import tpu as pltpu

def _chunk_rows(m):
    sizes = [128] + [240] * 8 if m == 2048 else [m // 8] * 8
    assert sum(sizes) == m
    return sizes


def kernel(x, pi):
    sizes = _chunk_rows(x.shape[1])
    offsets = [sum(sizes[:i]) for i in range(len(sizes))]
    n_chunks = len(sizes)
    x = pltpu.with_memory_space_constraint(x, pltpu.MemorySpace.HBM)
    pi = pltpu.with_memory_space_constraint(pi, pltpu.MemorySpace.HBM)

    def body(pi_hbm, x_hbm, out_ref, pi_smem, x_vmem, send_buf, pi_sem,
             local_sems, send_sems, recv_sems):
        my_x = lax.axis_index("x")
        my_y = lax.axis_index("y")
        my_z = lax.axis_index("z")
        other_y = 1 - my_y

        pi_copy = pltpu.make_async_copy(pi_hbm, pi_smem, pi_sem)
        pi_copy.start()

        local_copies = []
        for c in range(n_chunks):
            sl = pl.ds(offsets[c], sizes[c])
            cp = pltpu.make_async_copy(
                x_hbm.at[:, sl, :], x_vmem.at[:, sl, :], local_sems.at[c]
            )
            cp.start()
            local_copies.append(cp)

        barrier = pltpu.get_barrier_semaphore()
        pl.semaphore_signal(
            barrier,
            inc=1,
            device_id=(my_x, other_y, my_z),
            device_id_type=pl.DeviceIdType.MESH,
        )
        pl.semaphore_wait(barrier, 1)

        pi_copy.wait()
        dst_y = pi_smem[my_y]

        @pl.when(dst_y == my_y)
        def _():
            for c in range(n_chunks):
                local_copies[c].wait()
            send_buf[...] = x_vmem[...].astype(jnp.bfloat16)
            copy = pltpu.make_async_copy(send_buf, out_ref, send_sems.at[0])
            copy.start()
            copy.wait()

        @pl.when(dst_y != my_y)
        def _():
            rdmas = []
            for c in range(n_chunks):
                sl = pl.ds(offsets[c], sizes[c])
                local_copies[c].wait()
                send_buf[0, sl, :] = x_vmem[0, sl, :].astype(jnp.bfloat16)
                rdma = pltpu.make_async_remote_copy(
                    src_ref=send_buf.at[:, sl, :],
                    dst_ref=out_ref.at[:, sl, :],
                    send_sem=send_sems.at[c],
                    recv_sem=recv_sems.at[c],
                    device_id=(my_x, dst_y, my_z),
                    device_id_type=pl.DeviceIdType.MESH,
                )
                rdma.start()
                rdmas.append(rdma)
            for rdma in rdmas:
                rdma.wait()

    return pl.pallas_call(
        body,
        out_shape=jax.ShapeDtypeStruct(x.shape, jnp.bfloat16),
        in_specs=[
            pl.BlockSpec(memory_space=pl.ANY),
            pl.BlockSpec(memory_space=pl.ANY),
        ],
        out_specs=pl.BlockSpec(memory_space=pl.ANY),
        scratch_shapes=[
            pltpu.SMEM(pi.shape, pi.dtype),
            pltpu.VMEM(x.shape, x.dtype),
            pltpu.VMEM(x.shape, jnp.bfloat16),
            pltpu.SemaphoreType.DMA,
            pltpu.SemaphoreType.DMA((n_chunks,)),
            pltpu.SemaphoreType.DMA((n_chunks,)),
            pltpu.SemaphoreType.DMA((n_chunks,)),
        ],
        compiler_params=pltpu.CompilerParams(collective_id=0),
    )(pi, x)
